# baseline (speedup 1.0000x reference)
import functools

import jax
import jax.numpy as jnp
from jax import lax
from jax.experimental import pallas as pl
from jax.experimental.pallas import tpu as pltpu
from jax.experimental.pallas import tpu_sc as plsc

N = 10000
E = 320000
DN = 128
DE = 16
H = 64
HP = 128
MLPH = 128
L = 3
OUT = 2

_SC_CORES = 2
_SC_SUBCORES = 16
_GW = 128
NPAD = 10240
_ROWS_PER_SUB = NPAD // _SC_SUBCORES

_C = 1
_EC = E // _C
_BE = 2000
_BN = 2000


def _vec_mesh():
    return plsc.VectorSubcoreMesh(core_axis_name="core", subcore_axis_name="subcore")


def _sc_gather2(h, dst2d, src2d, c):
    i_off = c * (_EC // _GW)

    @functools.partial(
        pl.kernel,
        out_type=(
            jax.ShapeDtypeStruct((_EC, HP), jnp.float32),
            jax.ShapeDtypeStruct((_EC, HP), jnp.float32),
        ),
        mesh=_vec_mesh(),
    )
    def k(h_hbm, dst_hbm, src_hbm, xi_hbm, xj_hbm):
        def body(d_vmem, s_vmem, xi_vmem, xj_vmem):
            pltpu.sync_copy(h_hbm.at[d_vmem.at[0]], xi_vmem)
            pltpu.sync_copy(h_hbm.at[s_vmem.at[0]], xj_vmem)

        pltpu.emit_pipeline(
            body,
            grid=(_EC // _GW,),
            in_specs=[
                pl.BlockSpec((1, _GW), lambda i: (0, i + i_off)),
                pl.BlockSpec((1, _GW), lambda i: (0, i + i_off)),
            ],
            out_specs=[
                pl.BlockSpec((_GW, HP), lambda i: (i, 0)),
                pl.BlockSpec((_GW, HP), lambda i: (i, 0)),
            ],
            core_axis_name=("core", "subcore"),
            dimension_semantics=(pltpu.PARALLEL,),
        )(dst_hbm, src_hbm, xi_hbm, xj_hbm)

    return k(h, dst2d, src2d)


def _sc_scatter_add(msg, dst2d, zeros_hbm, c):
    i_off = c * (_EC // _GW)

    @functools.partial(
        pl.kernel,
        out_type=jax.ShapeDtypeStruct((_SC_CORES, NPAD, HP), jnp.float32),
        mesh=_vec_mesh(),
        scratch_types=[pltpu.VMEM_SHARED((NPAD, HP), jnp.float32)],
    )
    def k(msg_hbm, dst_hbm, z_hbm, out_hbm, acc):
        cid = lax.axis_index("core")
        sid = lax.axis_index("subcore")
        row0 = sid * _ROWS_PER_SUB
        pltpu.sync_copy(z_hbm.at[pl.ds(row0, _ROWS_PER_SUB)],
                        acc.at[pl.ds(row0, _ROWS_PER_SUB)])
        plsc.subcore_barrier()

        def body(m_vmem, d_vmem):
            pltpu.sync_copy(m_vmem, acc.at[d_vmem.at[0]], add=True)

        pltpu.emit_pipeline(
            body,
            grid=(_EC // _GW,),
            in_specs=[
                pl.BlockSpec((_GW, HP), lambda i: (i, 0)),
                pl.BlockSpec((1, _GW), lambda i: (0, i + i_off)),
            ],
            out_specs=[],
            core_axis_name=("core", "subcore"),
            dimension_semantics=(pltpu.PARALLEL,),
        )(msg_hbm, dst_hbm)

        plsc.subcore_barrier()
        pltpu.sync_copy(acc.at[pl.ds(row0, _ROWS_PER_SUB)],
                        out_hbm.at[cid, pl.ds(row0, _ROWS_PER_SUB)])

    return k(msg, dst2d, zeros_hbm)


def _node_encoder_body(x_ref, w1_ref, b1_ref, w2p_ref, b2p_ref, o_ref):
    t = jnp.dot(x_ref[...], w1_ref[...], preferred_element_type=jnp.float32)
    t = jnp.maximum(t + b1_ref[...], 0.0)
    o_ref[...] = jnp.dot(t, w2p_ref[...], preferred_element_type=jnp.float32) + b2p_ref[...]


def _tc_node_encoder(x, w1, b1, w2p, b2p):
    g = N // _BN
    return pl.pallas_call(
        _node_encoder_body,
        grid=(g,),
        in_specs=[
            pl.BlockSpec((_BN, DN), lambda i: (i, 0)),
            pl.BlockSpec((DN, H), lambda i: (0, 0)),
            pl.BlockSpec((1, H), lambda i: (0, 0)),
            pl.BlockSpec((H, HP), lambda i: (0, 0)),
            pl.BlockSpec((1, HP), lambda i: (0, 0)),
        ],
        out_specs=pl.BlockSpec((_BN, HP), lambda i: (i, 0)),
        out_shape=jax.ShapeDtypeStruct((N, HP), jnp.float32),
    )(x, w1, b1, w2p, b2p)


def _edge_encoder_body(a_ref, w1_ref, b1_ref, w2_ref, b2_ref, o_ref):
    t = jnp.dot(a_ref[...], w1_ref[...], preferred_element_type=jnp.float32)
    t = jnp.maximum(t + b1_ref[...], 0.0)
    o_ref[...] = jnp.dot(t, w2_ref[...], preferred_element_type=jnp.float32) + b2_ref[...]


def _tc_edge_encoder(edge_attr, w1, b1, w2, b2, c):
    g = _EC // _BE
    b_off = c * g
    return pl.pallas_call(
        _edge_encoder_body,
        grid=(g,),
        in_specs=[
            pl.BlockSpec((_BE, DE), lambda i: (i + b_off, 0)),
            pl.BlockSpec((DE, H), lambda i: (0, 0)),
            pl.BlockSpec((1, H), lambda i: (0, 0)),
            pl.BlockSpec((H, H), lambda i: (0, 0)),
            pl.BlockSpec((1, H), lambda i: (0, 0)),
        ],
        out_specs=pl.BlockSpec((_BE, H), lambda i: (i, 0)),
        out_shape=jax.ShapeDtypeStruct((_EC, H), jnp.float32),
    )(edge_attr, w1, b1, w2, b2)


def _edge_layer_body(xi_ref, xj_ref, ea_ref, emWxi_ref, emWxj_ref, emWea_ref,
                     emb1_ref, emW2_ref, emb2_ref, nmWxi_ref, nmWea_ref,
                     nmb1_ref, nmW2p_ref, nmb2p_ref, g_ref, b_ref,
                     msg_ref, eaout_ref):
    bf = jnp.bfloat16
    xi = xi_ref[...]
    xib = xi.astype(bf)
    xjb = xj_ref[...].astype(bf)
    ea = ea_ref[...]
    eab = ea.astype(bf)
    hmid = (jnp.dot(xib, emWxi_ref[...], preferred_element_type=jnp.float32)
            + jnp.dot(xjb, emWxj_ref[...], preferred_element_type=jnp.float32)
            + jnp.dot(eab, emWea_ref[...], preferred_element_type=jnp.float32)
            + emb1_ref[...])
    hmid = jnp.maximum(hmid, 0.0).astype(bf)
    ea_new = ea + jnp.dot(hmid, emW2_ref[...], preferred_element_type=jnp.float32) + emb2_ref[...]
    nmid = (jnp.dot(xib, nmWxi_ref[...], preferred_element_type=jnp.float32)
            + jnp.dot(ea_new.astype(bf), nmWea_ref[...], preferred_element_type=jnp.float32)
            + nmb1_ref[...])
    nmid = jnp.maximum(nmid, 0.0).astype(bf)
    msg_ref[...] = xi + jnp.dot(nmid, nmW2p_ref[...], preferred_element_type=jnp.float32) + nmb2p_ref[...]
    ea2 = ea + ea_new
    m = jnp.mean(ea2, axis=-1, keepdims=True)
    v = jnp.mean((ea2 - m) ** 2, axis=-1, keepdims=True)
    eaout_ref[...] = (ea2 - m) * lax.rsqrt(v + 1e-5) * g_ref[...] + b_ref[...]


def _tc_edge_layer(xi, xj, ea, emWxi, emWxj, emWea, emb1, emW2, emb2,
                   nmWxi, nmWea, nmb1, nmW2p, nmb2p, g, b):
    grid = _EC // _BE
    wspec = lambda r, c: pl.BlockSpec((r, c), lambda i: (0, 0))
    return pl.pallas_call(
        _edge_layer_body,
        grid=(grid,),
        in_specs=[
            pl.BlockSpec((_BE, HP), lambda i: (i, 0)),
            pl.BlockSpec((_BE, HP), lambda i: (i, 0)),
            pl.BlockSpec((_BE, H), lambda i: (i, 0)),
            wspec(HP, MLPH),
            wspec(HP, MLPH),
            wspec(H, MLPH),
            wspec(1, MLPH),
            wspec(MLPH, H),
            wspec(1, H),
            wspec(HP, MLPH),
            wspec(H, MLPH),
            wspec(1, MLPH),
            wspec(MLPH, HP),
            wspec(1, HP),
            wspec(1, H),
            wspec(1, H),
        ],
        out_specs=(
            pl.BlockSpec((_BE, HP), lambda i: (i, 0)),
            pl.BlockSpec((_BE, H), lambda i: (i, 0)),
        ),
        out_shape=(
            jax.ShapeDtypeStruct((_EC, HP), jnp.float32),
            jax.ShapeDtypeStruct((_EC, H), jnp.float32),
        ),
    )(xi, xj, ea, emWxi, emWxj, emWea, emb1, emW2, emb2,
      nmWxi, nmWea, nmb1, nmW2p, nmb2p, g, b)


def _node_update_body(h_ref, *rest):
    aggs = rest[:-3]
    g_ref, b_ref, o_ref = rest[-3:]
    t = h_ref[...]
    for a in aggs:
        t = t + a[0]
    t = t[:, :H]
    m = jnp.mean(t, axis=-1, keepdims=True)
    v = jnp.mean((t - m) ** 2, axis=-1, keepdims=True)
    res = (t - m) * lax.rsqrt(v + 1e-5) * g_ref[...] + b_ref[...]
    o_ref[...] = jnp.concatenate([res, jnp.zeros_like(res)], axis=1)


def _tc_node_update(h, aggs, g, b):
    grid = N // _BN
    agg_specs = []
    agg_args = []
    for a in aggs:
        for core in range(_SC_CORES):
            agg_specs.append(
                pl.BlockSpec((1, _BN, HP),
                             functools.partial(lambda core, i: (core, i, 0), core)))
            agg_args.append(a)
    return pl.pallas_call(
        _node_update_body,
        grid=(grid,),
        in_specs=[pl.BlockSpec((_BN, HP), lambda i: (i, 0))] + agg_specs + [
            pl.BlockSpec((1, H), lambda i: (0, 0)),
            pl.BlockSpec((1, H), lambda i: (0, 0)),
        ],
        out_specs=pl.BlockSpec((_BN, HP), lambda i: (i, 0)),
        out_shape=jax.ShapeDtypeStruct((N, HP), jnp.float32),
    )(h, *agg_args, g, b)


def _decoder_body(h_ref, w1p_ref, b1_ref, w2p_ref, b2p_ref, o_ref):
    t = jnp.dot(h_ref[...], w1p_ref[...], preferred_element_type=jnp.float32)
    t = jnp.maximum(t + b1_ref[...], 0.0)
    o_ref[...] = jnp.dot(t, w2p_ref[...], preferred_element_type=jnp.float32) + b2p_ref[...]


def _tc_decoder(h, w1p, b1, w2p, b2p):
    grid = N // _BN
    return pl.pallas_call(
        _decoder_body,
        grid=(grid,),
        in_specs=[
            pl.BlockSpec((_BN, HP), lambda i: (i, 0)),
            pl.BlockSpec((HP, H), lambda i: (0, 0)),
            pl.BlockSpec((1, H), lambda i: (0, 0)),
            pl.BlockSpec((H, 128), lambda i: (0, 0)),
            pl.BlockSpec((1, 128), lambda i: (0, 0)),
        ],
        out_specs=pl.BlockSpec((_BN, 128), lambda i: (i, 0)),
        out_shape=jax.ShapeDtypeStruct((N, 128), jnp.float32),
    )(h, w1p, b1, w2p, b2p)


def _pad_rows(w, rows):
    return jnp.zeros((rows, w.shape[1]), jnp.float32).at[: w.shape[0]].set(w)


def _pad_cols(w, cols):
    return jnp.zeros((w.shape[0], cols), jnp.float32).at[:, : w.shape[1]].set(w)


def kernel(x, edge_attr, edge_index, ne_W1, ne_b1, ne_W2, ne_b2, ee_W1, ee_b1,
           ee_W2, ee_b2, em_W1, em_b1, em_W2, em_b2, nm_W1, nm_b1, nm_W2,
           nm_b2, xln_g, xln_b, eln_g, eln_b, dec_W1, dec_b1, dec_W2, dec_b2):
    src2d = edge_index[0].reshape(1, E)
    dst2d = edge_index[1].reshape(1, E)

    r1 = lambda v: v.reshape(1, -1)
    bf = jnp.bfloat16
    zeros_nh = jnp.zeros((NPAD, HP), jnp.float32)

    ne_W2p = _pad_cols(ne_W2, HP)
    ne_b2p = _pad_cols(r1(ne_b2), HP)
    emWxi = _pad_rows(em_W1[0:H], HP).astype(bf)
    emWxj = _pad_rows(em_W1[H:2 * H], HP).astype(bf)
    emWea = em_W1[2 * H:3 * H].astype(bf)
    emW2b = em_W2.astype(bf)
    nmWxi = _pad_rows(nm_W1[0:H], HP).astype(bf)
    nmWea = nm_W1[H:2 * H].astype(bf)
    nmW2p = _pad_cols(nm_W2, HP).astype(bf)
    nmb2p = _pad_cols(r1(nm_b2), HP)
    decW1p = _pad_rows(dec_W1, HP)
    decW2p = _pad_cols(dec_W2, 128)
    decb2p = _pad_cols(r1(dec_b2), 128)

    h = _tc_node_encoder(x, ne_W1, r1(ne_b1), ne_W2p, ne_b2p)
    ea = [_tc_edge_encoder(edge_attr, ee_W1, r1(ee_b1), ee_W2, r1(ee_b2), c)
          for c in range(_C)]

    for i in range(L):
        aggs = []
        for c in range(_C):
            xi, xj = _sc_gather2(h, dst2d, src2d, c)
            msg, ea[c] = _tc_edge_layer(
                xi, xj, ea[c], emWxi, emWxj, emWea, r1(em_b1), emW2b,
                r1(em_b2), nmWxi, nmWea, r1(nm_b1), nmW2p, nmb2p,
                r1(eln_g[i]), r1(eln_b[i]))
            aggs.append(_sc_scatter_add(msg, dst2d, zeros_nh, c))
        h = _tc_node_update(h, aggs, r1(xln_g[i]), r1(xln_b[i]))

    outp = _tc_decoder(h, decW1p, r1(dec_b1), decW2p, decb2p)
    return outp[:, :OUT]


# speedup vs baseline: 1.1011x; 1.1011x over previous
import functools

import jax
import jax.numpy as jnp
from jax import lax
from jax.experimental import pallas as pl
from jax.experimental.pallas import tpu as pltpu
from jax.experimental.pallas import tpu_sc as plsc

N = 10000
E = 320000
DN = 128
DE = 16
H = 64
HP = 128
MLPH = 128
L = 3
OUT = 2

_SC_CORES = 2
_SC_SUBCORES = 16
_GW = 128
NPAD = 10240
_ROWS_PER_SUB = NPAD // _SC_SUBCORES

_C = 4
_EC = E // _C
_BE = 2000
_BN = 2000


def _vec_mesh():
    return plsc.VectorSubcoreMesh(core_axis_name="core", subcore_axis_name="subcore")


def _sc_gather2(h, dst2d, src2d, c):
    i_off = c * (_EC // _GW)

    @functools.partial(
        pl.kernel,
        out_type=(
            jax.ShapeDtypeStruct((_EC, HP), jnp.float32),
            jax.ShapeDtypeStruct((_EC, HP), jnp.float32),
        ),
        mesh=_vec_mesh(),
    )
    def k(h_hbm, dst_hbm, src_hbm, xi_hbm, xj_hbm):
        def body(d_vmem, s_vmem, xi_vmem, xj_vmem):
            pltpu.sync_copy(h_hbm.at[d_vmem.at[0]], xi_vmem)
            pltpu.sync_copy(h_hbm.at[s_vmem.at[0]], xj_vmem)

        pltpu.emit_pipeline(
            body,
            grid=(_EC // _GW,),
            in_specs=[
                pl.BlockSpec((1, _GW), lambda i: (0, i + i_off)),
                pl.BlockSpec((1, _GW), lambda i: (0, i + i_off)),
            ],
            out_specs=[
                pl.BlockSpec((_GW, HP), lambda i: (i, 0)),
                pl.BlockSpec((_GW, HP), lambda i: (i, 0)),
            ],
            core_axis_name=("core", "subcore"),
            dimension_semantics=(pltpu.PARALLEL,),
        )(dst_hbm, src_hbm, xi_hbm, xj_hbm)

    return k(h, dst2d, src2d)


def _sc_scatter_add(msg, dst2d, zeros_hbm, c):
    i_off = c * (_EC // _GW)

    @functools.partial(
        pl.kernel,
        out_type=jax.ShapeDtypeStruct((_SC_CORES, NPAD, HP), jnp.float32),
        mesh=_vec_mesh(),
        scratch_types=[pltpu.VMEM_SHARED((NPAD, HP), jnp.float32)],
    )
    def k(msg_hbm, dst_hbm, z_hbm, out_hbm, acc):
        cid = lax.axis_index("core")
        sid = lax.axis_index("subcore")
        row0 = sid * _ROWS_PER_SUB
        pltpu.sync_copy(z_hbm.at[pl.ds(row0, _ROWS_PER_SUB)],
                        acc.at[pl.ds(row0, _ROWS_PER_SUB)])
        plsc.subcore_barrier()

        def body(m_vmem, d_vmem):
            pltpu.sync_copy(m_vmem, acc.at[d_vmem.at[0]], add=True)

        pltpu.emit_pipeline(
            body,
            grid=(_EC // _GW,),
            in_specs=[
                pl.BlockSpec((_GW, HP), lambda i: (i, 0)),
                pl.BlockSpec((1, _GW), lambda i: (0, i + i_off)),
            ],
            out_specs=[],
            core_axis_name=("core", "subcore"),
            dimension_semantics=(pltpu.PARALLEL,),
        )(msg_hbm, dst_hbm)

        plsc.subcore_barrier()
        pltpu.sync_copy(acc.at[pl.ds(row0, _ROWS_PER_SUB)],
                        out_hbm.at[cid, pl.ds(row0, _ROWS_PER_SUB)])

    return k(msg, dst2d, zeros_hbm)


def _node_encoder_body(x_ref, w1_ref, b1_ref, w2p_ref, b2p_ref, o_ref):
    t = jnp.dot(x_ref[...], w1_ref[...], preferred_element_type=jnp.float32)
    t = jnp.maximum(t + b1_ref[...], 0.0)
    o_ref[...] = jnp.dot(t, w2p_ref[...], preferred_element_type=jnp.float32) + b2p_ref[...]


def _tc_node_encoder(x, w1, b1, w2p, b2p):
    g = N // _BN
    return pl.pallas_call(
        _node_encoder_body,
        grid=(g,),
        in_specs=[
            pl.BlockSpec((_BN, DN), lambda i: (i, 0)),
            pl.BlockSpec((DN, H), lambda i: (0, 0)),
            pl.BlockSpec((1, H), lambda i: (0, 0)),
            pl.BlockSpec((H, HP), lambda i: (0, 0)),
            pl.BlockSpec((1, HP), lambda i: (0, 0)),
        ],
        out_specs=pl.BlockSpec((_BN, HP), lambda i: (i, 0)),
        out_shape=jax.ShapeDtypeStruct((N, HP), jnp.float32),
    )(x, w1, b1, w2p, b2p)


def _edge_encoder_body(a_ref, w1_ref, b1_ref, w2_ref, b2_ref, o_ref):
    t = jnp.dot(a_ref[...], w1_ref[...], preferred_element_type=jnp.float32)
    t = jnp.maximum(t + b1_ref[...], 0.0)
    o_ref[...] = jnp.dot(t, w2_ref[...], preferred_element_type=jnp.float32) + b2_ref[...]


def _tc_edge_encoder(edge_attr, w1, b1, w2, b2, c):
    g = _EC // _BE
    b_off = c * g
    return pl.pallas_call(
        _edge_encoder_body,
        grid=(g,),
        in_specs=[
            pl.BlockSpec((_BE, DE), lambda i: (i + b_off, 0)),
            pl.BlockSpec((DE, H), lambda i: (0, 0)),
            pl.BlockSpec((1, H), lambda i: (0, 0)),
            pl.BlockSpec((H, H), lambda i: (0, 0)),
            pl.BlockSpec((1, H), lambda i: (0, 0)),
        ],
        out_specs=pl.BlockSpec((_BE, H), lambda i: (i, 0)),
        out_shape=jax.ShapeDtypeStruct((_EC, H), jnp.float32),
    )(edge_attr, w1, b1, w2, b2)


def _edge_layer_body(xi_ref, xj_ref, ea_ref, emWxi_ref, emWxj_ref, emWea_ref,
                     emb1_ref, emW2_ref, emb2_ref, nmWxi_ref, nmWea_ref,
                     nmb1_ref, nmW2p_ref, nmb2p_ref, g_ref, b_ref,
                     msg_ref, eaout_ref):
    bf = jnp.bfloat16
    xi = xi_ref[...]
    xib = xi.astype(bf)
    xjb = xj_ref[...].astype(bf)
    ea = ea_ref[...]
    eab = ea.astype(bf)
    hmid = (jnp.dot(xib, emWxi_ref[...], preferred_element_type=jnp.float32)
            + jnp.dot(xjb, emWxj_ref[...], preferred_element_type=jnp.float32)
            + jnp.dot(eab, emWea_ref[...], preferred_element_type=jnp.float32)
            + emb1_ref[...])
    hmid = jnp.maximum(hmid, 0.0).astype(bf)
    ea_new = ea + jnp.dot(hmid, emW2_ref[...], preferred_element_type=jnp.float32) + emb2_ref[...]
    nmid = (jnp.dot(xib, nmWxi_ref[...], preferred_element_type=jnp.float32)
            + jnp.dot(ea_new.astype(bf), nmWea_ref[...], preferred_element_type=jnp.float32)
            + nmb1_ref[...])
    nmid = jnp.maximum(nmid, 0.0).astype(bf)
    msg_ref[...] = xi + jnp.dot(nmid, nmW2p_ref[...], preferred_element_type=jnp.float32) + nmb2p_ref[...]
    ea2 = ea + ea_new
    m = jnp.mean(ea2, axis=-1, keepdims=True)
    v = jnp.mean((ea2 - m) ** 2, axis=-1, keepdims=True)
    eaout_ref[...] = (ea2 - m) * lax.rsqrt(v + 1e-5) * g_ref[...] + b_ref[...]


def _tc_edge_layer(xi, xj, ea, emWxi, emWxj, emWea, emb1, emW2, emb2,
                   nmWxi, nmWea, nmb1, nmW2p, nmb2p, g, b):
    grid = _EC // _BE
    wspec = lambda r, c: pl.BlockSpec((r, c), lambda i: (0, 0))
    return pl.pallas_call(
        _edge_layer_body,
        grid=(grid,),
        in_specs=[
            pl.BlockSpec((_BE, HP), lambda i: (i, 0)),
            pl.BlockSpec((_BE, HP), lambda i: (i, 0)),
            pl.BlockSpec((_BE, H), lambda i: (i, 0)),
            wspec(HP, MLPH),
            wspec(HP, MLPH),
            wspec(H, MLPH),
            wspec(1, MLPH),
            wspec(MLPH, H),
            wspec(1, H),
            wspec(HP, MLPH),
            wspec(H, MLPH),
            wspec(1, MLPH),
            wspec(MLPH, HP),
            wspec(1, HP),
            wspec(1, H),
            wspec(1, H),
        ],
        out_specs=(
            pl.BlockSpec((_BE, HP), lambda i: (i, 0)),
            pl.BlockSpec((_BE, H), lambda i: (i, 0)),
        ),
        out_shape=(
            jax.ShapeDtypeStruct((_EC, HP), jnp.float32),
            jax.ShapeDtypeStruct((_EC, H), jnp.float32),
        ),
    )(xi, xj, ea, emWxi, emWxj, emWea, emb1, emW2, emb2,
      nmWxi, nmWea, nmb1, nmW2p, nmb2p, g, b)


def _node_update_body(h_ref, *rest):
    aggs = rest[:-3]
    g_ref, b_ref, o_ref = rest[-3:]
    t = h_ref[...]
    for a in aggs:
        t = t + a[0]
    t = t[:, :H]
    m = jnp.mean(t, axis=-1, keepdims=True)
    v = jnp.mean((t - m) ** 2, axis=-1, keepdims=True)
    res = (t - m) * lax.rsqrt(v + 1e-5) * g_ref[...] + b_ref[...]
    o_ref[...] = jnp.concatenate([res, jnp.zeros_like(res)], axis=1)


def _tc_node_update(h, aggs, g, b):
    grid = N // _BN
    agg_specs = []
    agg_args = []
    for a in aggs:
        for core in range(_SC_CORES):
            agg_specs.append(
                pl.BlockSpec((1, _BN, HP),
                             functools.partial(lambda core, i: (core, i, 0), core)))
            agg_args.append(a)
    return pl.pallas_call(
        _node_update_body,
        grid=(grid,),
        in_specs=[pl.BlockSpec((_BN, HP), lambda i: (i, 0))] + agg_specs + [
            pl.BlockSpec((1, H), lambda i: (0, 0)),
            pl.BlockSpec((1, H), lambda i: (0, 0)),
        ],
        out_specs=pl.BlockSpec((_BN, HP), lambda i: (i, 0)),
        out_shape=jax.ShapeDtypeStruct((N, HP), jnp.float32),
    )(h, *agg_args, g, b)


def _decoder_body(h_ref, w1p_ref, b1_ref, w2p_ref, b2p_ref, o_ref):
    t = jnp.dot(h_ref[...], w1p_ref[...], preferred_element_type=jnp.float32)
    t = jnp.maximum(t + b1_ref[...], 0.0)
    o_ref[...] = jnp.dot(t, w2p_ref[...], preferred_element_type=jnp.float32) + b2p_ref[...]


def _tc_decoder(h, w1p, b1, w2p, b2p):
    grid = N // _BN
    return pl.pallas_call(
        _decoder_body,
        grid=(grid,),
        in_specs=[
            pl.BlockSpec((_BN, HP), lambda i: (i, 0)),
            pl.BlockSpec((HP, H), lambda i: (0, 0)),
            pl.BlockSpec((1, H), lambda i: (0, 0)),
            pl.BlockSpec((H, 128), lambda i: (0, 0)),
            pl.BlockSpec((1, 128), lambda i: (0, 0)),
        ],
        out_specs=pl.BlockSpec((_BN, 128), lambda i: (i, 0)),
        out_shape=jax.ShapeDtypeStruct((N, 128), jnp.float32),
    )(h, w1p, b1, w2p, b2p)


def _pad_rows(w, rows):
    return jnp.zeros((rows, w.shape[1]), jnp.float32).at[: w.shape[0]].set(w)


def _pad_cols(w, cols):
    return jnp.zeros((w.shape[0], cols), jnp.float32).at[:, : w.shape[1]].set(w)


def kernel(x, edge_attr, edge_index, ne_W1, ne_b1, ne_W2, ne_b2, ee_W1, ee_b1,
           ee_W2, ee_b2, em_W1, em_b1, em_W2, em_b2, nm_W1, nm_b1, nm_W2,
           nm_b2, xln_g, xln_b, eln_g, eln_b, dec_W1, dec_b1, dec_W2, dec_b2):
    src2d = edge_index[0].reshape(1, E)
    dst2d = edge_index[1].reshape(1, E)

    r1 = lambda v: v.reshape(1, -1)
    bf = jnp.bfloat16
    zeros_nh = jnp.zeros((NPAD, HP), jnp.float32)

    ne_W2p = _pad_cols(ne_W2, HP)
    ne_b2p = _pad_cols(r1(ne_b2), HP)
    emWxi = _pad_rows(em_W1[0:H], HP).astype(bf)
    emWxj = _pad_rows(em_W1[H:2 * H], HP).astype(bf)
    emWea = em_W1[2 * H:3 * H].astype(bf)
    emW2b = em_W2.astype(bf)
    nmWxi = _pad_rows(nm_W1[0:H], HP).astype(bf)
    nmWea = nm_W1[H:2 * H].astype(bf)
    nmW2p = _pad_cols(nm_W2, HP).astype(bf)
    nmb2p = _pad_cols(r1(nm_b2), HP)
    decW1p = _pad_rows(dec_W1, HP)
    decW2p = _pad_cols(dec_W2, 128)
    decb2p = _pad_cols(r1(dec_b2), 128)

    h = _tc_node_encoder(x, ne_W1, r1(ne_b1), ne_W2p, ne_b2p)
    ea = [_tc_edge_encoder(edge_attr, ee_W1, r1(ee_b1), ee_W2, r1(ee_b2), c)
          for c in range(_C)]

    for i in range(L):
        aggs = []
        for c in range(_C):
            xi, xj = _sc_gather2(h, dst2d, src2d, c)
            msg, ea[c] = _tc_edge_layer(
                xi, xj, ea[c], emWxi, emWxj, emWea, r1(em_b1), emW2b,
                r1(em_b2), nmWxi, nmWea, r1(nm_b1), nmW2p, nmb2p,
                r1(eln_g[i]), r1(eln_b[i]))
            aggs.append(_sc_scatter_add(msg, dst2d, zeros_nh, c))
        h = _tc_node_update(h, aggs, r1(xln_g[i]), r1(xln_b[i]))

    outp = _tc_decoder(h, decW1p, r1(dec_b1), decW2p, decb2p)
    return outp[:, :OUT]
